# baseline (speedup 1.0000x reference)
import functools

import jax
import jax.numpy as jnp
from jax import lax
from jax.experimental import pallas as pl
from jax.experimental.pallas import tpu as pltpu
from jax.experimental.pallas import tpu_sc as plsc

_B, _C, _H, _W = 4, 384, 32, 32
_E = 8
_FF = 768
_N = _B * _H * _W
_PAIRS = 2 * _N
_BLK = 256
_NBLK = (_PAIRS + _E * _BLK) // _BLK
_P = _NBLK * _BLK

_NW = 32
_GCH = 64


def _router_body(x_ref, wg_ref, pos0_ref, pos1_ref, w0_ref, w1_ref, be_ref):
    x = x_ref[...]
    logits = jnp.dot(x, wg_ref[...], preferred_element_type=jnp.float32)
    z = logits - jnp.max(logits, axis=1, keepdims=True)
    ez = jnp.exp(z)
    probs = ez / jnp.sum(ez, axis=1, keepdims=True)

    lane = lax.broadcasted_iota(jnp.int32, probs.shape, 1)
    m1 = jnp.max(probs, axis=1, keepdims=True)
    i1 = jnp.min(jnp.where(probs == m1, lane, _E), axis=1, keepdims=True)
    p2 = jnp.where(lane == i1, -1e30, probs)
    m2 = jnp.max(p2, axis=1, keepdims=True)
    i2 = jnp.min(jnp.where(p2 == m2, lane, _E), axis=1, keepdims=True)
    s = m1 + m2
    w0_ref[...] = (m1 / s)[:, 0]
    w1_ref[...] = (m2 / s)[:, 0]

    c = (lane == i1).astype(jnp.float32) + (lane == i2).astype(jnp.float32)

    cb_rows = 128
    row = lax.broadcasted_iota(jnp.int32, (cb_rows, cb_rows), 0)
    col = lax.broadcasted_iota(jnp.int32, (cb_rows, cb_rows), 1)
    lstrict = (row > col).astype(jnp.float32)
    carry = jnp.zeros((1, _E), jnp.float32)
    cum_blocks = []
    for b in range(_N // cb_rows):
        cb = c[b * cb_rows:(b + 1) * cb_rows]
        cum_blocks.append(
            jnp.dot(lstrict, cb, preferred_element_type=jnp.float32) + carry)
        carry = carry + jnp.sum(cb, axis=0, keepdims=True)
    cum = jnp.concatenate(cum_blocks, axis=0)
    counts = carry

    pad = jnp.ceil(counts / _BLK) * _BLK
    r8 = lax.broadcasted_iota(jnp.int32, (_E, _E), 0)
    c8 = lax.broadcasted_iota(jnp.int32, (_E, _E), 1)
    up = (r8 < c8).astype(jnp.float32)
    off = jnp.dot(pad, up, preferred_element_type=jnp.float32)
    off_incl = off + pad
    total = jnp.sum(pad)

    tgt = off + cum
    pos0_ref[...] = jnp.sum(jnp.where(lane == i1, tgt, 0.0), axis=1).astype(jnp.int32)
    pos1_ref[...] = jnp.sum(jnp.where(lane == i2, tgt, 0.0), axis=1).astype(jnp.int32)

    bstart = (lax.broadcasted_iota(jnp.int32, (_NBLK, _E), 0) * _BLK).astype(jnp.float32)
    becount = jnp.sum((off_incl <= bstart).astype(jnp.float32), axis=1)
    be = jnp.minimum(becount, float(_E - 1))
    live = bstart[:, 0] < total
    be_ref[...] = jnp.where(live, be, -1.0).astype(jnp.int32)


def _router(t, Wg):
    return pl.pallas_call(
        _router_body,
        out_shape=[
            jax.ShapeDtypeStruct((_N,), jnp.int32),
            jax.ShapeDtypeStruct((_N,), jnp.int32),
            jax.ShapeDtypeStruct((_N,), jnp.float32),
            jax.ShapeDtypeStruct((_N,), jnp.float32),
            jax.ShapeDtypeStruct((_NBLK,), jnp.int32),
        ],
    )(t, Wg)


def _dispatch_body(t_hbm, pos0_hbm, pos1_hbm, w0_hbm, w1_hbm, xs_out, ws_out,
                   rowid_sp, zb_v, ia0, ia1, ib0, ib1, ta_v, tb_v,
                   wa0, wa1, wb0, wb1, idxa, idxb, idxc, buf_a, buf_b,
                   sem, sem_o):
    cid = lax.axis_index("c")
    sid = lax.axis_index("s")
    wid = sid * 2 + cid
    nsc = 16

    zseg = _P // nsc

    def zinit(i, _):
        zb_v[pl.ds(i * 16, 16)] = jnp.zeros((16,), jnp.int32)
        return 0
    lax.fori_loop(0, zseg // 16, zinit, 0)
    pltpu.sync_copy(zb_v, rowid_sp.at[pl.ds(sid * zseg, zseg)])
    plsc.subcore_barrier()

    tseg = _N // nsc
    tb0 = sid * tseg
    pltpu.sync_copy(pos0_hbm.at[pl.ds(tb0, 128)], ia0)
    pltpu.sync_copy(pos0_hbm.at[pl.ds(tb0 + 128, 128)], ia1)
    pltpu.sync_copy(pos1_hbm.at[pl.ds(tb0, 128)], ib0)
    pltpu.sync_copy(pos1_hbm.at[pl.ds(tb0 + 128, 128)], ib1)

    def tinit(i, _):
        ta_v[pl.ds(i * 16, 16)] = lax.iota(jnp.int32, 16) + (tb0 + i * 16)
        tb_v[pl.ds(i * 16, 16)] = lax.iota(jnp.int32, 16) + (tb0 + 128 + i * 16)
        return 0
    lax.fori_loop(0, 8, tinit, 0)
    s1 = pltpu.async_copy(ta_v, rowid_sp.at[ia0], sem)
    s2 = pltpu.async_copy(tb_v, rowid_sp.at[ia1], sem)
    s3 = pltpu.async_copy(ta_v, rowid_sp.at[ib0], sem)
    s4 = pltpu.async_copy(tb_v, rowid_sp.at[ib1], sem)
    s1.wait()
    s2.wait()
    s3.wait()
    s4.wait()

    @pl.when(cid == 0)
    def _():
        pltpu.sync_copy(w0_hbm.at[pl.ds(tb0, 128)], wa0)
        pltpu.sync_copy(w0_hbm.at[pl.ds(tb0 + 128, 128)], wa1)
        pltpu.sync_copy(w1_hbm.at[pl.ds(tb0, 128)], wb0)
        pltpu.sync_copy(w1_hbm.at[pl.ds(tb0 + 128, 128)], wb1)
        q1 = pltpu.async_copy(wa0, ws_out.at[ia0], sem_o)
        q2 = pltpu.async_copy(wa1, ws_out.at[ia1], sem_o)
        q3 = pltpu.async_copy(wb0, ws_out.at[ib0], sem_o)
        q4 = pltpu.async_copy(wb1, ws_out.at[ib1], sem_o)
        q1.wait()
        q2.wait()
        q3.wait()
        q4.wait()
    plsc.subcore_barrier()

    base = wid * (_P // _NW)
    pltpu.sync_copy(rowid_sp.at[pl.ds(base, 128)], idxa)
    pltpu.sync_copy(rowid_sp.at[pl.ds(base + 128, 128)], idxb)
    pltpu.sync_copy(rowid_sp.at[pl.ds(base + 256, 64)], idxc)
    g1 = pltpu.async_copy(t_hbm.at[idxa], buf_a, sem)
    g2 = pltpu.async_copy(t_hbm.at[idxb], buf_b, sem)
    g1.wait()
    o1 = pltpu.async_copy(buf_a, xs_out.at[pl.ds(base, 128)], sem_o)
    g2.wait()
    o2 = pltpu.async_copy(buf_b, xs_out.at[pl.ds(base + 128, 128)], sem_o)
    o1.wait()
    g3 = pltpu.async_copy(t_hbm.at[idxc], buf_a.at[pl.ds(0, 64)], sem)
    g3.wait()
    o3 = pltpu.async_copy(buf_a.at[pl.ds(0, 64)],
                          xs_out.at[pl.ds(base + 256, 64)], sem_o)
    o2.wait()
    o3.wait()


def _sc_dispatch(t, pos0, pos1, w0, w1):
    mesh = plsc.VectorSubcoreMesh(core_axis_name="c", subcore_axis_name="s")
    return pl.kernel(
        _dispatch_body,
        mesh=mesh,
        out_type=[
            jax.ShapeDtypeStruct((_P, _C), jnp.float32),
            jax.ShapeDtypeStruct((_P,), jnp.float32),
        ],
        scratch_types=[
            pltpu.VMEM_SHARED((_P,), jnp.int32),
            pltpu.VMEM((_P // 16,), jnp.int32),
            pltpu.VMEM((128,), jnp.int32),
            pltpu.VMEM((128,), jnp.int32),
            pltpu.VMEM((128,), jnp.int32),
            pltpu.VMEM((128,), jnp.int32),
            pltpu.VMEM((128,), jnp.int32),
            pltpu.VMEM((128,), jnp.int32),
            pltpu.VMEM((128,), jnp.float32),
            pltpu.VMEM((128,), jnp.float32),
            pltpu.VMEM((128,), jnp.float32),
            pltpu.VMEM((128,), jnp.float32),
            pltpu.VMEM((128,), jnp.int32),
            pltpu.VMEM((128,), jnp.int32),
            pltpu.VMEM((64,), jnp.int32),
            pltpu.VMEM((128, _C), jnp.float32),
            pltpu.VMEM((128, _C), jnp.float32),
            pltpu.SemaphoreType.DMA,
            pltpu.SemaphoreType.DMA,
        ],
    )(t, pos0, pos1, w0, w1)


def _grouped_body(be_ref, x_ref, wg_ref, wu_ref, wd_ref, w_ref, o_ref):
    b = pl.program_id(0)

    @pl.when(be_ref[b] >= 0)
    def _():
        e = jnp.maximum(be_ref[b], 0)
        x = x_ref[...]
        g = jnp.dot(x, wg_ref[pl.ds(e, 1)][0],
                    preferred_element_type=jnp.float32)
        u = jnp.dot(x, wu_ref[pl.ds(e, 1)][0],
                    preferred_element_type=jnp.float32)
        h = g * jax.nn.sigmoid(g) * u
        o = jnp.dot(h, wd_ref[pl.ds(e, 1)][0],
                    preferred_element_type=jnp.float32)
        o_ref[...] = o * w_ref[...]


def _grouped(be, xs, W_gate, W_up, W_down, ws2d):
    grid_spec = pltpu.PrefetchScalarGridSpec(
        num_scalar_prefetch=1,
        grid=(_NBLK,),
        in_specs=[
            pl.BlockSpec((_BLK, _C), lambda b, be: (b, 0)),
            pl.BlockSpec((_E, _C, _FF), lambda b, be: (0, 0, 0)),
            pl.BlockSpec((_E, _C, _FF), lambda b, be: (0, 0, 0)),
            pl.BlockSpec((_E, _FF, _C), lambda b, be: (0, 0, 0)),
            pl.BlockSpec((_BLK, 1), lambda b, be: (b, 0)),
        ],
        out_specs=pl.BlockSpec((_BLK, _C), lambda b, be: (b, 0)),
    )
    return pl.pallas_call(
        _grouped_body,
        grid_spec=grid_spec,
        out_shape=jax.ShapeDtypeStruct((_P, _C), jnp.float32),
    )(be, xs, W_gate, W_up, W_down, ws2d)


def _combine_body(ys_hbm, pos0_hbm, pos1_hbm, ya_out, yb_out,
                  idx0_v, idx1_v, buf_a, buf_b, sem, sem_o):
    cid = lax.axis_index("c")
    sid = lax.axis_index("s")
    wid = sid * 2 + cid
    per = _N // _NW
    start = wid * per
    pltpu.sync_copy(pos0_hbm.at[pl.ds(start, per)], idx0_v)
    pltpu.sync_copy(pos1_hbm.at[pl.ds(start, per)], idx1_v)
    g1 = pltpu.async_copy(ys_hbm.at[idx0_v], buf_a, sem)
    g2 = pltpu.async_copy(ys_hbm.at[idx1_v], buf_b, sem)
    g1.wait()
    o1 = pltpu.async_copy(buf_a, ya_out.at[pl.ds(start, per)], sem_o)
    g2.wait()
    o2 = pltpu.async_copy(buf_b, yb_out.at[pl.ds(start, per)], sem_o)
    o1.wait()
    o2.wait()


def _sc_combine(ys, pos0, pos1):
    mesh = plsc.VectorSubcoreMesh(core_axis_name="c", subcore_axis_name="s")
    return pl.kernel(
        _combine_body,
        mesh=mesh,
        out_type=[
            jax.ShapeDtypeStruct((_N, _C), jnp.float32),
            jax.ShapeDtypeStruct((_N, _C), jnp.float32),
        ],
        scratch_types=[
            pltpu.VMEM((_N // _NW,), jnp.int32),
            pltpu.VMEM((_N // _NW,), jnp.int32),
            pltpu.VMEM((_N // _NW, _C), jnp.float32),
            pltpu.VMEM((_N // _NW, _C), jnp.float32),
            pltpu.SemaphoreType.DMA,
            pltpu.SemaphoreType.DMA,
        ],
    )(ys, pos0, pos1)


_STB = 512

def _shared_body(x_ref, wsg_ref, wsu_ref, wsd_ref, ya_ref, yb_ref, o_ref):
    x = x_ref[...]
    g = jnp.dot(x, wsg_ref[...], preferred_element_type=jnp.float32)
    u = jnp.dot(x, wsu_ref[...], preferred_element_type=jnp.float32)
    h = g * jax.nn.sigmoid(g) * u
    o = jnp.dot(h, wsd_ref[...], preferred_element_type=jnp.float32)
    o_ref[...] = o + ya_ref[...] + yb_ref[...]


def _shared(t, Ws_gate, Ws_up, Ws_down, ya, yb):
    return pl.pallas_call(
        _shared_body,
        grid=(_N // _STB,),
        in_specs=[
            pl.BlockSpec((_STB, _C), lambda i: (i, 0)),
            pl.BlockSpec((_C, _FF), lambda i: (0, 0)),
            pl.BlockSpec((_C, _FF), lambda i: (0, 0)),
            pl.BlockSpec((_FF, _C), lambda i: (0, 0)),
            pl.BlockSpec((_STB, _C), lambda i: (i, 0)),
            pl.BlockSpec((_STB, _C), lambda i: (i, 0)),
        ],
        out_specs=pl.BlockSpec((_STB, _C), lambda i: (i, 0)),
        out_shape=jax.ShapeDtypeStruct((_N, _C), jnp.float32),
    )(t, Ws_gate, Ws_up, Ws_down, ya, yb)


def kernel(x, Wg, W_gate, W_up, W_down, Ws_gate, Ws_up, Ws_down):
    b, c, h, w = x.shape
    t = jnp.transpose(x, (0, 2, 3, 1)).reshape(b * h * w, c)
    pos0, pos1, w0, w1, be = _router(t, Wg)
    xs, ws = _sc_dispatch(t, pos0, pos1, w0, w1)
    ys = _grouped(be, xs, W_gate, W_up, W_down, ws.reshape(_P, 1))
    ya, yb = _sc_combine(ys, pos0, pos1)
    y = _shared(t, Ws_gate, Ws_up, Ws_down, ya, yb)
    return jnp.transpose(y.reshape(b, h, w, c), (0, 3, 1, 2))


# speedup vs baseline: 1.4169x; 1.4169x over previous
import functools

import jax
import jax.numpy as jnp
from jax import lax
from jax.experimental import pallas as pl
from jax.experimental.pallas import tpu as pltpu
from jax.experimental.pallas import tpu_sc as plsc

_B, _C, _H, _W = 4, 384, 32, 32
_E = 8
_FF = 768
_N = _B * _H * _W
_PAIRS = 2 * _N
_BLK = 256
_NBLK = (_PAIRS + _E * _BLK) // _BLK
_P = _NBLK * _BLK

_NW = 32
_GCH = 64


def _router_body(x_ref, wg_ref, pos0_ref, pos1_ref, w0_ref, w1_ref, be_ref):
    x = x_ref[...]
    logits = jnp.dot(x, wg_ref[...], preferred_element_type=jnp.float32)
    z = logits - jnp.max(logits, axis=1, keepdims=True)
    ez = jnp.exp(z)
    probs = ez / jnp.sum(ez, axis=1, keepdims=True)

    lane = lax.broadcasted_iota(jnp.int32, probs.shape, 1)
    m1 = jnp.max(probs, axis=1, keepdims=True)
    i1 = jnp.min(jnp.where(probs == m1, lane, _E), axis=1, keepdims=True)
    p2 = jnp.where(lane == i1, -1e30, probs)
    m2 = jnp.max(p2, axis=1, keepdims=True)
    i2 = jnp.min(jnp.where(p2 == m2, lane, _E), axis=1, keepdims=True)
    s = m1 + m2
    w0_ref[...] = (m1 / s)[:, 0]
    w1_ref[...] = (m2 / s)[:, 0]

    c = (lane == i1).astype(jnp.float32) + (lane == i2).astype(jnp.float32)

    cb_rows = 128
    row = lax.broadcasted_iota(jnp.int32, (cb_rows, cb_rows), 0)
    col = lax.broadcasted_iota(jnp.int32, (cb_rows, cb_rows), 1)
    lstrict = (row > col).astype(jnp.float32)
    carry = jnp.zeros((1, _E), jnp.float32)
    cum_blocks = []
    for b in range(_N // cb_rows):
        cb = c[b * cb_rows:(b + 1) * cb_rows]
        cum_blocks.append(
            jnp.dot(lstrict, cb, preferred_element_type=jnp.float32) + carry)
        carry = carry + jnp.sum(cb, axis=0, keepdims=True)
    cum = jnp.concatenate(cum_blocks, axis=0)
    counts = carry

    pad = jnp.ceil(counts / _BLK) * _BLK
    r8 = lax.broadcasted_iota(jnp.int32, (_E, _E), 0)
    c8 = lax.broadcasted_iota(jnp.int32, (_E, _E), 1)
    up = (r8 < c8).astype(jnp.float32)
    off = jnp.dot(pad, up, preferred_element_type=jnp.float32)
    off_incl = off + pad
    total = jnp.sum(pad)

    tgt = off + cum
    pos0_ref[...] = jnp.sum(jnp.where(lane == i1, tgt, 0.0), axis=1).astype(jnp.int32)
    pos1_ref[...] = jnp.sum(jnp.where(lane == i2, tgt, 0.0), axis=1).astype(jnp.int32)

    bstart = (lax.broadcasted_iota(jnp.int32, (_NBLK, _E), 0) * _BLK).astype(jnp.float32)
    becount = jnp.sum((off_incl <= bstart).astype(jnp.float32), axis=1)
    be = jnp.minimum(becount, float(_E - 1))
    live = bstart[:, 0] < total
    be_ref[...] = jnp.where(live, be, -1.0).astype(jnp.int32)


def _router(t, Wg):
    return pl.pallas_call(
        _router_body,
        out_shape=[
            jax.ShapeDtypeStruct((_N,), jnp.int32),
            jax.ShapeDtypeStruct((_N,), jnp.int32),
            jax.ShapeDtypeStruct((_N,), jnp.float32),
            jax.ShapeDtypeStruct((_N,), jnp.float32),
            jax.ShapeDtypeStruct((_NBLK,), jnp.int32),
        ],
    )(t, Wg)


def _dispatch_body(t_hbm, pos0_hbm, pos1_hbm, w0_hbm, w1_hbm, xs_out, ws_out,
                   rowid_sp, zb_v, ia0, ia1, ib0, ib1, ta_v, tb_v,
                   wa0, wa1, wb0, wb1, idxa, idxb, idxc, buf_a, buf_b,
                   sem, sem_o):
    cid = lax.axis_index("c")
    sid = lax.axis_index("s")
    wid = sid * 2 + cid
    nsc = 16

    zseg = _P // nsc

    def zinit(i, _):
        zb_v[pl.ds(i * 16, 16)] = jnp.bitwise_and(
            lax.iota(jnp.int32, 16) + (sid * zseg + i * 16), _N - 1)
        return 0
    lax.fori_loop(0, zseg // 16, zinit, 0)
    pltpu.sync_copy(zb_v, rowid_sp.at[pl.ds(sid * zseg, zseg)])
    plsc.subcore_barrier()

    tseg = _N // nsc
    tb0 = sid * tseg
    pltpu.sync_copy(pos0_hbm.at[pl.ds(tb0, 128)], ia0)
    pltpu.sync_copy(pos0_hbm.at[pl.ds(tb0 + 128, 128)], ia1)
    pltpu.sync_copy(pos1_hbm.at[pl.ds(tb0, 128)], ib0)
    pltpu.sync_copy(pos1_hbm.at[pl.ds(tb0 + 128, 128)], ib1)

    def tinit(i, _):
        ta_v[pl.ds(i * 16, 16)] = lax.iota(jnp.int32, 16) + (tb0 + i * 16)
        tb_v[pl.ds(i * 16, 16)] = lax.iota(jnp.int32, 16) + (tb0 + 128 + i * 16)
        return 0
    lax.fori_loop(0, 8, tinit, 0)
    s1 = pltpu.async_copy(ta_v, rowid_sp.at[ia0], sem)
    s2 = pltpu.async_copy(tb_v, rowid_sp.at[ia1], sem)
    s3 = pltpu.async_copy(ta_v, rowid_sp.at[ib0], sem)
    s4 = pltpu.async_copy(tb_v, rowid_sp.at[ib1], sem)
    s1.wait()
    s2.wait()
    s3.wait()
    s4.wait()

    @pl.when(cid == 0)
    def _():
        pltpu.sync_copy(w0_hbm.at[pl.ds(tb0, 128)], wa0)
        pltpu.sync_copy(w0_hbm.at[pl.ds(tb0 + 128, 128)], wa1)
        pltpu.sync_copy(w1_hbm.at[pl.ds(tb0, 128)], wb0)
        pltpu.sync_copy(w1_hbm.at[pl.ds(tb0 + 128, 128)], wb1)
        q1 = pltpu.async_copy(wa0, ws_out.at[ia0], sem_o)
        q2 = pltpu.async_copy(wa1, ws_out.at[ia1], sem_o)
        q3 = pltpu.async_copy(wb0, ws_out.at[ib0], sem_o)
        q4 = pltpu.async_copy(wb1, ws_out.at[ib1], sem_o)
        q1.wait()
        q2.wait()
        q3.wait()
        q4.wait()
    plsc.subcore_barrier()

    base = wid * (_P // _NW)
    pltpu.sync_copy(rowid_sp.at[pl.ds(base, 128)], idxa)
    pltpu.sync_copy(rowid_sp.at[pl.ds(base + 128, 128)], idxb)
    pltpu.sync_copy(rowid_sp.at[pl.ds(base + 256, 64)], idxc)
    g1 = pltpu.async_copy(t_hbm.at[idxa], buf_a, sem)
    g2 = pltpu.async_copy(t_hbm.at[idxb], buf_b, sem)
    g1.wait()
    o1 = pltpu.async_copy(buf_a, xs_out.at[pl.ds(base, 128)], sem_o)
    g2.wait()
    o2 = pltpu.async_copy(buf_b, xs_out.at[pl.ds(base + 128, 128)], sem_o)
    o1.wait()
    g3 = pltpu.async_copy(t_hbm.at[idxc], buf_a.at[pl.ds(0, 64)], sem)
    g3.wait()
    o3 = pltpu.async_copy(buf_a.at[pl.ds(0, 64)],
                          xs_out.at[pl.ds(base + 256, 64)], sem_o)
    o2.wait()
    o3.wait()


def _sc_dispatch(t, pos0, pos1, w0, w1):
    mesh = plsc.VectorSubcoreMesh(core_axis_name="c", subcore_axis_name="s")
    return pl.kernel(
        _dispatch_body,
        mesh=mesh,
        out_type=[
            jax.ShapeDtypeStruct((_P, _C), jnp.float32),
            jax.ShapeDtypeStruct((_P,), jnp.float32),
        ],
        scratch_types=[
            pltpu.VMEM_SHARED((_P,), jnp.int32),
            pltpu.VMEM((_P // 16,), jnp.int32),
            pltpu.VMEM((128,), jnp.int32),
            pltpu.VMEM((128,), jnp.int32),
            pltpu.VMEM((128,), jnp.int32),
            pltpu.VMEM((128,), jnp.int32),
            pltpu.VMEM((128,), jnp.int32),
            pltpu.VMEM((128,), jnp.int32),
            pltpu.VMEM((128,), jnp.float32),
            pltpu.VMEM((128,), jnp.float32),
            pltpu.VMEM((128,), jnp.float32),
            pltpu.VMEM((128,), jnp.float32),
            pltpu.VMEM((128,), jnp.int32),
            pltpu.VMEM((128,), jnp.int32),
            pltpu.VMEM((64,), jnp.int32),
            pltpu.VMEM((128, _C), jnp.float32),
            pltpu.VMEM((128, _C), jnp.float32),
            pltpu.SemaphoreType.DMA,
            pltpu.SemaphoreType.DMA,
        ],
    )(t, pos0, pos1, w0, w1)


def _grouped_body(be_ref, x_ref, wg_ref, wu_ref, wd_ref, w_ref, o_ref):
    b = pl.program_id(0)

    @pl.when(be_ref[b] >= 0)
    def _():
        e = jnp.maximum(be_ref[b], 0)
        x = x_ref[...]
        g = jnp.dot(x, wg_ref[pl.ds(e, 1)][0],
                    preferred_element_type=jnp.float32)
        u = jnp.dot(x, wu_ref[pl.ds(e, 1)][0],
                    preferred_element_type=jnp.float32)
        h = g * jax.nn.sigmoid(g) * u
        o = jnp.dot(h, wd_ref[pl.ds(e, 1)][0],
                    preferred_element_type=jnp.float32)
        o_ref[...] = o * w_ref[...]


def _grouped(be, xs, W_gate, W_up, W_down, ws2d):
    grid_spec = pltpu.PrefetchScalarGridSpec(
        num_scalar_prefetch=1,
        grid=(_NBLK,),
        in_specs=[
            pl.BlockSpec((_BLK, _C), lambda b, be: (b, 0)),
            pl.BlockSpec((_E, _C, _FF), lambda b, be: (0, 0, 0)),
            pl.BlockSpec((_E, _C, _FF), lambda b, be: (0, 0, 0)),
            pl.BlockSpec((_E, _FF, _C), lambda b, be: (0, 0, 0)),
            pl.BlockSpec((_BLK, 1), lambda b, be: (b, 0)),
        ],
        out_specs=pl.BlockSpec((_BLK, _C), lambda b, be: (b, 0)),
    )
    return pl.pallas_call(
        _grouped_body,
        grid_spec=grid_spec,
        out_shape=jax.ShapeDtypeStruct((_P, _C), jnp.float32),
    )(be, xs, W_gate, W_up, W_down, ws2d)


def _combine_body(ys_hbm, pos0_hbm, pos1_hbm, ya_out, yb_out,
                  idx0_v, idx1_v, buf_a, buf_b, sem, sem_o):
    cid = lax.axis_index("c")
    sid = lax.axis_index("s")
    wid = sid * 2 + cid
    per = _N // _NW
    start = wid * per
    pltpu.sync_copy(pos0_hbm.at[pl.ds(start, per)], idx0_v)
    pltpu.sync_copy(pos1_hbm.at[pl.ds(start, per)], idx1_v)
    g1 = pltpu.async_copy(ys_hbm.at[idx0_v], buf_a, sem)
    g2 = pltpu.async_copy(ys_hbm.at[idx1_v], buf_b, sem)
    g1.wait()
    o1 = pltpu.async_copy(buf_a, ya_out.at[pl.ds(start, per)], sem_o)
    g2.wait()
    o2 = pltpu.async_copy(buf_b, yb_out.at[pl.ds(start, per)], sem_o)
    o1.wait()
    o2.wait()


def _sc_combine(ys, pos0, pos1):
    mesh = plsc.VectorSubcoreMesh(core_axis_name="c", subcore_axis_name="s")
    return pl.kernel(
        _combine_body,
        mesh=mesh,
        out_type=[
            jax.ShapeDtypeStruct((_N, _C), jnp.float32),
            jax.ShapeDtypeStruct((_N, _C), jnp.float32),
        ],
        scratch_types=[
            pltpu.VMEM((_N // _NW,), jnp.int32),
            pltpu.VMEM((_N // _NW,), jnp.int32),
            pltpu.VMEM((_N // _NW, _C), jnp.float32),
            pltpu.VMEM((_N // _NW, _C), jnp.float32),
            pltpu.SemaphoreType.DMA,
            pltpu.SemaphoreType.DMA,
        ],
    )(ys, pos0, pos1)


_STB = 512

def _shared_body(x_ref, wsg_ref, wsu_ref, wsd_ref, ya_ref, yb_ref, o_ref):
    x = x_ref[...]
    g = jnp.dot(x, wsg_ref[...], preferred_element_type=jnp.float32)
    u = jnp.dot(x, wsu_ref[...], preferred_element_type=jnp.float32)
    h = g * jax.nn.sigmoid(g) * u
    o = jnp.dot(h, wsd_ref[...], preferred_element_type=jnp.float32)
    o_ref[...] = o + ya_ref[...] + yb_ref[...]


def _shared(t, Ws_gate, Ws_up, Ws_down, ya, yb):
    return pl.pallas_call(
        _shared_body,
        grid=(_N // _STB,),
        in_specs=[
            pl.BlockSpec((_STB, _C), lambda i: (i, 0)),
            pl.BlockSpec((_C, _FF), lambda i: (0, 0)),
            pl.BlockSpec((_C, _FF), lambda i: (0, 0)),
            pl.BlockSpec((_FF, _C), lambda i: (0, 0)),
            pl.BlockSpec((_STB, _C), lambda i: (i, 0)),
            pl.BlockSpec((_STB, _C), lambda i: (i, 0)),
        ],
        out_specs=pl.BlockSpec((_STB, _C), lambda i: (i, 0)),
        out_shape=jax.ShapeDtypeStruct((_N, _C), jnp.float32),
    )(t, Ws_gate, Ws_up, Ws_down, ya, yb)


def kernel(x, Wg, W_gate, W_up, W_down, Ws_gate, Ws_up, Ws_down):
    b, c, h, w = x.shape
    t = jnp.transpose(x, (0, 2, 3, 1)).reshape(b * h * w, c)
    pos0, pos1, w0, w1, be = _router(t, Wg)
    xs, ws = _sc_dispatch(t, pos0, pos1, w0, w1)
    ys = _grouped(be, xs, W_gate, W_up, W_down, ws.reshape(_P, 1))
    ya, yb = _sc_combine(ys, pos0, pos1)
    y = _shared(t, Ws_gate, Ws_up, Ws_down, ya, yb)
    return jnp.transpose(y.reshape(b, h, w, c), (0, 3, 1, 2))
